# baseline (speedup 1.0000x reference)
import functools

import jax
import jax.numpy as jnp
from jax import lax
from jax.experimental import pallas as pl
from jax.experimental.pallas import tpu as pltpu
from jax.experimental.pallas import tpu_sc as plsc

_NW = 32
_CHUNK = 4096


def _make_gather(total_rows: int, d: int):
    rows_per_w = total_rows // _NW
    n_chunks = rows_per_w // _CHUNK
    mesh = plsc.VectorSubcoreMesh(core_axis_name="c", subcore_axis_name="s")

    @functools.partial(
        pl.kernel,
        mesh=mesh,
        out_type=jax.ShapeDtypeStruct((total_rows, d), jnp.float32),
        scratch_types=[
            pltpu.VMEM((_CHUNK,), jnp.int32),
            pltpu.VMEM((_CHUNK, d), jnp.float32),
            pltpu.SemaphoreType.DMA,
        ],
        compiler_params=pltpu.CompilerParams(use_tc_tiling_on_sc=False),
    )
    def gather_k(table_hbm, idx_hbm, out_hbm, idx_v, rows_v, sem):
        wid = lax.axis_index("s") * 2 + lax.axis_index("c")
        base = wid * rows_per_w

        def body(i, carry):
            off = base + i * _CHUNK
            pltpu.sync_copy(idx_hbm.at[pl.ds(off, _CHUNK)], idx_v)
            pltpu.async_copy(table_hbm.at[idx_v], rows_v, sem).wait()
            pltpu.sync_copy(rows_v, out_hbm.at[pl.ds(off, _CHUNK)])
            return carry

        lax.fori_loop(0, n_chunks, body, 0)

    return gather_k


def kernel(particles, log_weights):
    t, n, d = particles.shape
    key = jax.random.key(42)
    keys = jax.random.split(key, t)

    def prep(lw, k):
        w = jnp.exp(lw)
        _, subkey = jax.random.split(k)
        p_cuml = jnp.cumsum(w)
        r = p_cuml[-1] * (1 - jax.random.uniform(subkey, (n,), dtype=p_cuml.dtype))
        return p_cuml, r

    p_cuml, r = jax.vmap(prep)(log_weights, keys)
    ind = jax.vmap(jnp.searchsorted)(p_cuml, r).astype(jnp.int32)
    flat_idx = (jnp.arange(t, dtype=jnp.int32)[:, None] * n + ind).reshape(-1)
    flat_parts = particles.reshape(t * n, d)
    out = _make_gather(t * n, d)(flat_parts, flat_idx)
    return out.reshape(t, n, d)


# speedup vs baseline: 2.3076x; 2.3076x over previous
import functools

import jax
import jax.numpy as jnp
from jax import lax
from jax.experimental import pallas as pl
from jax.experimental.pallas import tpu as pltpu
from jax.experimental.pallas import tpu_sc as plsc

_NW = 32
_L = 16
_CHUNK = 2048
_FINE = 16
_COARSE_LVLS = 13
_FINE_LVLS = 4


def _make_resample(t_steps: int, n: int, d: int):
    n_coarse = n // _FINE
    n_chunks = n // _CHUNK
    vregs = _CHUNK // _L
    mesh = plsc.VectorSubcoreMesh(core_axis_name="c", subcore_axis_name="s")

    @functools.partial(
        pl.kernel,
        mesh=mesh,
        out_type=jax.ShapeDtypeStruct((t_steps * n, d), jnp.float32),
        scratch_types=[
            pltpu.VMEM((n_coarse,), jnp.float32),
            pltpu.VMEM((_CHUNK,), jnp.float32),
            pltpu.VMEM((_CHUNK,), jnp.int32),
            pltpu.VMEM((_CHUNK, _FINE), jnp.float32),
            pltpu.VMEM((_CHUNK,), jnp.int32),
            pltpu.VMEM((_CHUNK, d), jnp.float32),
            pltpu.SemaphoreType.DMA,
        ],
        compiler_params=pltpu.CompilerParams(
            use_tc_tiling_on_sc=False, needs_layout_passes=False),
    )
    def resample_k(coarse_hbm, q_hbm, cumrows_hbm, parts_hbm, out_hbm,
                   coarse_v, q_v, rid_v, rows_v, pid_v, prow_v, sem):
        wid = lax.axis_index("s") * 2 + lax.axis_index("c")
        iota = lax.broadcasted_iota(jnp.int32, (_L,), 0)

        pltpu.sync_copy(coarse_hbm.at[pl.ds(wid * n_coarse, n_coarse)],
                        coarse_v)

        def chunk_body(ci, carry):
            off = wid * n + ci * _CHUNK
            pltpu.sync_copy(q_hbm.at[pl.ds(off, _CHUNK)], q_v)

            def coarse_body(j, c):
                q = q_v[pl.ds(j * _L, _L)]
                cnt = jnp.zeros((_L,), jnp.int32)
                for lvl in range(_COARSE_LVLS):
                    step = 1 << (_COARSE_LVLS - 1 - lvl)
                    vals = plsc.load_gather(coarse_v, [cnt + (step - 1)])
                    cnt = cnt + jnp.where(vals < q, step, 0)
                rid_v[pl.ds(j * _L, _L)] = wid * n_coarse + cnt
                return c

            lax.fori_loop(0, vregs, coarse_body, 0)

            pltpu.async_copy(cumrows_hbm.at[rid_v], rows_v, sem).wait()

            def fine_body(j, c):
                q = q_v[pl.ds(j * _L, _L)]
                rid = rid_v[pl.ds(j * _L, _L)]
                row = j * _L + iota
                cnt = jnp.zeros((_L,), jnp.int32)
                for lvl in range(_FINE_LVLS):
                    step = 1 << (_FINE_LVLS - 1 - lvl)
                    vals = plsc.load_gather(rows_v, [row, cnt + (step - 1)])
                    cnt = cnt + jnp.where(vals < q, step, 0)
                pid_v[pl.ds(j * _L, _L)] = rid * _FINE + cnt
                return c

            lax.fori_loop(0, vregs, fine_body, 0)

            pltpu.async_copy(parts_hbm.at[pid_v], prow_v, sem).wait()
            pltpu.sync_copy(prow_v, out_hbm.at[pl.ds(off, _CHUNK)])
            return carry

        lax.fori_loop(0, n_chunks, chunk_body, 0)

    return resample_k


def kernel(particles, log_weights):
    t, n, d = particles.shape
    key = jax.random.key(42)
    keys = jax.random.split(key, t)

    def prep(lw, k):
        w = jnp.exp(lw)
        _, subkey = jax.random.split(k)
        p_cuml = jnp.cumsum(w)
        r = p_cuml[-1] * (1 - jax.random.uniform(subkey, (n,), dtype=p_cuml.dtype))
        return p_cuml, r

    p_cuml, r = jax.vmap(prep)(log_weights, keys)
    coarse = p_cuml[:, _FINE - 1::_FINE]
    cumrows = p_cuml.reshape(t * (n // _FINE), _FINE)
    out = _make_resample(t, n, d)(
        coarse.reshape(-1), r.reshape(-1), cumrows,
        particles.reshape(t * n, d))
    return out.reshape(t, n, d)
